# baseline (speedup 1.0000x reference)
import jax
import jax.numpy as jnp
from jax import lax
from jax.experimental import pallas as pl
from jax.experimental.pallas import tpu as pltpu
from jax.experimental.pallas import tpu_sc as plsc

N = 32768
D1 = 512
DY = 1000
DYA = 1008
NSEG = 1024
NC, NS, L = 2, 16, 16
NW = NC * NS
SEG_PER_W = NSEG // NW
DUMP = SEG_PER_W
CH = 16

BROW = 256


def _gate_body(x1_ref, x2_ref, w1_ref, b1_ref, g1_ref, be1_ref, g2_ref,
               be2_ref, a_ref):
    x1 = x1_ref[...]
    x2 = x2_ref[...]
    mu1 = jnp.mean(x1, axis=1, keepdims=True)
    v1 = jnp.mean((x1 - mu1) ** 2, axis=1, keepdims=True)
    x1n = (x1 - mu1) * lax.rsqrt(v1 + 1e-5) * g1_ref[...] + be1_ref[...]
    mu2 = jnp.mean(x2, axis=1, keepdims=True)
    v2 = jnp.mean((x2 - mu2) ** 2, axis=1, keepdims=True)
    x2n = (x2 - mu2) * lax.rsqrt(v2 + 1e-5) * g2_ref[...] + be2_ref[...]
    z = jnp.sum(x2n * w1_ref[...], axis=1, keepdims=True) + b1_ref[0, 0]
    gate = 1.0 / (1.0 + jnp.exp(-z))
    a_ref[...] = gate * x1n


def _gate_call(x1, x2, W1, b1, g1, be1, g2, be2):
    full = lambda i: (0, 0)
    return pl.pallas_call(
        _gate_body,
        grid=(N // BROW,),
        in_specs=[
            pl.BlockSpec((BROW, D1), lambda i: (i, 0)),
            pl.BlockSpec((BROW, D1), lambda i: (i, 0)),
            pl.BlockSpec((1, D1), full),
            pl.BlockSpec((1, 1), full),
            pl.BlockSpec((1, D1), full),
            pl.BlockSpec((1, D1), full),
            pl.BlockSpec((1, D1), full),
            pl.BlockSpec((1, D1), full),
        ],
        out_specs=pl.BlockSpec((BROW, D1), lambda i: (i, 0)),
        out_shape=jax.ShapeDtypeStruct((N, D1), jnp.float32),
    )(x1, x2, W1, b1, g1, be1, g2, be2)


def _make_segsum_body(dsrc, dacc, ch, with_count):
    def body(rows_hbm, lab_hbm, st_hbm, out_hbm,
             sbuf, lb0, lb1, rb0, rb1, acc, sem0, sem1):
        cid = lax.axis_index("c")
        sid = lax.axis_index("s")
        wid = cid * NS + sid
        iota = lax.iota(jnp.int32, L)
        zero16 = jnp.zeros((L,), jnp.float32)
        cnt_pat = (iota == 0).astype(jnp.float32)
        tcols = jnp.where(iota < 8, iota + DY, iota + (DY - L))

        def zrow(i, _):
            for j in range(dacc // L):
                acc[i, pl.ds(j * L, L)] = zero16
            return 0

        lax.fori_loop(0, SEG_PER_W + 1, zrow, 0)

        pltpu.sync_copy(st_hbm.at[pl.ds(wid * L, L)], sbuf)
        sv = sbuf[pl.ds(0, L)]
        lo = sv[0]
        hi = sv[1]
        lo8 = (lo // 8) * 8
        nchunks = (hi - lo8 + ch - 1) // ch

        def cbase(c):
            return jnp.minimum(lo8 + c * ch, N - ch)

        def start(c, lb, rb, sem):
            b = cbase(c)
            pltpu.async_copy(lab_hbm.at[pl.ds(b, ch)], lb, sem)
            pltpu.async_copy(rows_hbm.at[pl.ds(b, ch)], rb, sem)

        def drain(lb, rb, sem):
            pltpu.make_async_copy(lab_hbm.at[pl.ds(0, ch)], lb, sem).wait()
            pltpu.make_async_copy(rows_hbm.at[pl.ds(0, ch)], rb, sem).wait()

        def tree_sum(vs):
            while len(vs) > 1:
                vs = [vs[i] + vs[i + 1] for i in range(0, len(vs) - 1, 2)] \
                    + ([vs[-1]] if len(vs) % 2 else [])
            return vs[0]

        def process(c, lb, rb):
            b = cbase(c)
            grow_ok = lambda g: (g >= lo) & (g < hi) & (c < nchunks)
            for g0 in range(0, ch, L):
                u16 = lb[pl.ds(g0, L)]
                ok = grow_ok(b + g0 + iota)
                idx_eff = jnp.where(ok, u16 - wid * SEG_PER_W, DUMP)
                u0 = idx_eff[0]
                same = jnp.all(idx_eff == jnp.broadcast_to(u0, (L,)))

                @pl.when(same)
                def _fast():
                    for j in range(dsrc // L):
                        vs = [rb[g0 + t, pl.ds(j * L, L)] for t in range(L)]
                        plsc.addupdate(acc.at[u0, pl.ds(j * L, L)],
                                       tree_sum(vs))
                    if with_count:
                        row = jnp.broadcast_to(u0, (L,))
                        vs = [rb[g0 + t, pl.ds(DY - L, L)] for t in range(L)]
                        vals = jnp.where(iota < 8, float(L) * cnt_pat,
                                         tree_sum(vs))
                        plsc.addupdate_scatter(acc, [row, tcols], vals)

                @pl.when(jnp.logical_not(same))
                def _slow():
                    u_ts = [idx_eff[t] for t in range(L)]
                    for t16 in range(L):
                        t = g0 + t16
                        u_t = u_ts[t16]
                        NB = 8
                        prev = None
                        for i0 in range(0, dsrc // L, NB):
                            cur = [(j, rb[t, pl.ds(j * L, L)])
                                   for j in range(i0,
                                                  min(i0 + NB, dsrc // L))]
                            if prev is not None:
                                for j, v in prev:
                                    plsc.addupdate(
                                        acc.at[u_t, pl.ds(j * L, L)], v)
                            prev = cur
                        for j, v in prev:
                            plsc.addupdate(acc.at[u_t, pl.ds(j * L, L)], v)
                        if with_count:
                            row = jnp.broadcast_to(u_t, (L,))
                            v984 = rb[t, pl.ds(DY - L, L)]
                            vals = jnp.where(iota < 8, cnt_pat, v984)
                            plsc.addupdate_scatter(acc, [row, tcols], vals)

        npairs = jnp.maximum((nchunks + 1) // 2, 1)
        start(0, lb0, rb0, sem0)
        start(1, lb1, rb1, sem1)

        def pair(p, _):
            c0 = 2 * p
            drain(lb0, rb0, sem0)
            process(c0, lb0, rb0)
            start(c0 + 2, lb0, rb0, sem0)
            drain(lb1, rb1, sem1)
            process(c0 + 1, lb1, rb1)
            start(c0 + 3, lb1, rb1, sem1)
            return 0

        lax.fori_loop(0, npairs, pair, 0)
        drain(lb0, rb0, sem0)
        drain(lb1, rb1, sem1)

        out0 = wid * SEG_PER_W
        pltpu.sync_copy(acc.at[pl.ds(0, SEG_PER_W)],
                        out_hbm.at[pl.ds(out0, SEG_PER_W)])

    return body


def _segsum_call(rows, labels, starts, dsrc, dacc, ch, with_count):
    run = pl.kernel(
        _make_segsum_body(dsrc, dacc, ch, with_count),
        out_type=jax.ShapeDtypeStruct((NSEG, dacc), jnp.float32),
        mesh=plsc.VectorSubcoreMesh(
            core_axis_name="c", subcore_axis_name="s", num_cores=NC,
            num_subcores=NS),
        compiler_params=pltpu.CompilerParams(needs_layout_passes=False),
        scratch_types=[
            pltpu.VMEM((L,), jnp.int32),
            pltpu.VMEM((ch,), jnp.int32),
            pltpu.VMEM((ch,), jnp.int32),
            pltpu.VMEM((ch, dsrc), jnp.float32),
            pltpu.VMEM((ch, dsrc), jnp.float32),
            pltpu.VMEM((SEG_PER_W + 1, dacc), jnp.float32),
            pltpu.SemaphoreType.DMA,
            pltpu.SemaphoreType.DMA,
        ],
    )
    return run(rows, labels, starts)


def _final_body(suma_ref, sumy_ref, w2_ref, b2_ref, out_ref):
    sa = suma_ref[...]
    sy = sumy_ref[:, 0:DY]
    cnt = sumy_ref[:, DY:DY + 1]
    valid = cnt > 0.0
    safe = jnp.where(valid, cnt, 1.0)
    res = sa / safe
    logits = lax.dot_general(
        res, w2_ref[...], (((1,), (1,)), ((), ())),
        preferred_element_type=jnp.float32) + b2_ref[...]
    m = jnp.max(logits, axis=1, keepdims=True)
    e = jnp.exp(logits - m)
    p = e / jnp.sum(e, axis=1, keepdims=True)
    m2 = jnp.max(p, axis=1, keepdims=True)
    lse = jnp.log(jnp.sum(jnp.exp(p - m2), axis=1, keepdims=True)) + m2
    logp = p - lse
    per = jnp.sum(sy * logp, axis=1, keepdims=True) / safe
    per = jnp.where(valid, per, 0.0)
    u = jnp.sum(valid.astype(jnp.float32), axis=0, keepdims=True)
    out_ref[...] = -jnp.sum(per, axis=0, keepdims=True) / u


def _final_call(suma, sumy, W2, b2):
    full = lambda: (0, 0)
    return pl.pallas_call(
        _final_body,
        in_specs=[
            pl.BlockSpec((NSEG, D1), full),
            pl.BlockSpec((NSEG, DYA), full),
            pl.BlockSpec((DY, D1), full),
            pl.BlockSpec((1, DY), full),
        ],
        out_specs=pl.BlockSpec((1, 1), full),
        out_shape=jax.ShapeDtypeStruct((1, 1), jnp.float32),
    )(suma, sumy, W2, b2)


def kernel(x1, x2, y, W1, b1, W2, b2, g1, be1, g2, be2, labels):
    labels = labels.astype(jnp.int32)
    bounds = jnp.searchsorted(
        labels, jnp.arange(0, NSEG + 1, SEG_PER_W, dtype=jnp.int32)
    ).astype(jnp.int32)
    starts = jnp.zeros((NW, L), jnp.int32)
    starts = starts.at[:, 0].set(bounds[:-1]).at[:, 1].set(bounds[1:])
    starts = starts.reshape(NW * L)
    sumy = _segsum_call(y, labels, starts, DY, DYA, CH, True)
    a = _gate_call(
        x1, x2, W1, b1.reshape(1, 1), g1.reshape(1, D1), be1.reshape(1, D1),
        g2.reshape(1, D1), be2.reshape(1, D1))
    suma = _segsum_call(a, labels, starts, D1, D1, 2 * CH, False)
    out = _final_call(suma, sumy, W2, b2.reshape(1, DY))
    return out[0, 0]


# speedup vs baseline: 1.3335x; 1.3335x over previous
import jax
import jax.numpy as jnp
from jax import lax
from jax.experimental import pallas as pl
from jax.experimental.pallas import tpu as pltpu
from jax.experimental.pallas import tpu_sc as plsc

N = 32768
D1 = 512
DY = 1000
DYA = 1008
NSEG = 1024
NC, NS, L = 2, 16, 16
NW = NC * NS
SEG_PER_W = NSEG // NW
DUMP = SEG_PER_W
CH = 16

BROW = 256


def _gate_body(x1_ref, x2_ref, w1_ref, b1_ref, g1_ref, be1_ref, g2_ref,
               be2_ref, a_ref):
    x1 = x1_ref[...]
    x2 = x2_ref[...]
    mu1 = jnp.mean(x1, axis=1, keepdims=True)
    v1 = jnp.mean((x1 - mu1) ** 2, axis=1, keepdims=True)
    x1n = (x1 - mu1) * lax.rsqrt(v1 + 1e-5) * g1_ref[...] + be1_ref[...]
    mu2 = jnp.mean(x2, axis=1, keepdims=True)
    v2 = jnp.mean((x2 - mu2) ** 2, axis=1, keepdims=True)
    x2n = (x2 - mu2) * lax.rsqrt(v2 + 1e-5) * g2_ref[...] + be2_ref[...]
    z = jnp.sum(x2n * w1_ref[...], axis=1, keepdims=True) + b1_ref[0, 0]
    gate = 1.0 / (1.0 + jnp.exp(-z))
    a_ref[...] = gate * x1n


def _gate_call(x1, x2, W1, b1, g1, be1, g2, be2):
    full = lambda i: (0, 0)
    return pl.pallas_call(
        _gate_body,
        grid=(N // BROW,),
        in_specs=[
            pl.BlockSpec((BROW, D1), lambda i: (i, 0)),
            pl.BlockSpec((BROW, D1), lambda i: (i, 0)),
            pl.BlockSpec((1, D1), full),
            pl.BlockSpec((1, 1), full),
            pl.BlockSpec((1, D1), full),
            pl.BlockSpec((1, D1), full),
            pl.BlockSpec((1, D1), full),
            pl.BlockSpec((1, D1), full),
        ],
        out_specs=pl.BlockSpec((BROW, D1), lambda i: (i, 0)),
        out_shape=jax.ShapeDtypeStruct((N, D1), jnp.float32),
    )(x1, x2, W1, b1, g1, be1, g2, be2)


def _make_segsum_body(dsrc, dacc, ch, with_count):
    def body(rows_hbm, lab_hbm, st_hbm, out_hbm,
             sbuf, lb0, lb1, rb0, rb1, acc, sem0, sem1):
        cid = lax.axis_index("c")
        sid = lax.axis_index("s")
        wid = cid * NS + sid
        iota = lax.iota(jnp.int32, L)
        zero16 = jnp.zeros((L,), jnp.float32)
        cnt_pat = (iota == 0).astype(jnp.float32)
        tcols = jnp.where(iota < 8, iota + DY, iota + (DY - L))

        def zrow(i, _):
            for j in range(dacc // L):
                acc[i, pl.ds(j * L, L)] = zero16
            return 0

        lax.fori_loop(0, SEG_PER_W + 1, zrow, 0)

        pltpu.sync_copy(st_hbm.at[pl.ds(wid * L, L)], sbuf)
        sv = sbuf[pl.ds(0, L)]
        lo = sv[0]
        hi = sv[1]
        lo8 = (lo // 8) * 8
        nchunks = (hi - lo8 + ch - 1) // ch

        def cbase(c):
            return jnp.minimum(lo8 + c * ch, N - ch)

        def start(c, lb, rb, sem):
            b = cbase(c)
            pltpu.async_copy(lab_hbm.at[pl.ds(b, ch)], lb, sem)
            pltpu.async_copy(rows_hbm.at[pl.ds(b, ch)], rb, sem)

        def drain(lb, rb, sem):
            pltpu.make_async_copy(lab_hbm.at[pl.ds(0, ch)], lb, sem).wait()
            pltpu.make_async_copy(rows_hbm.at[pl.ds(0, ch)], rb, sem).wait()

        def process(c, lb, rb):
            b = cbase(c)
            grow_ok = lambda g: (g >= lo) & (g < hi) & (c < nchunks)
            for g0 in range(0, ch, L):
                u16 = lb[pl.ds(g0, L)]
                ok = grow_ok(b + g0 + iota)
                idx_eff = jnp.where(ok, u16 - wid * SEG_PER_W, DUMP)
                u_ts = [idx_eff[t] for t in range(L)]
                for t16 in range(L):
                    t = g0 + t16
                    u_t = u_ts[t16]
                    NB = 8
                    prev = None
                    for i0 in range(0, dsrc // L, NB):
                        cur = [(j, rb[t, pl.ds(j * L, L)])
                               for j in range(i0, min(i0 + NB, dsrc // L))]
                        if prev is not None:
                            for j, v in prev:
                                plsc.addupdate(acc.at[u_t, pl.ds(j * L, L)],
                                               v)
                        prev = cur
                    for j, v in prev:
                        plsc.addupdate(acc.at[u_t, pl.ds(j * L, L)], v)
                    if with_count:
                        row = jnp.broadcast_to(u_t, (L,))
                        v984 = rb[t, pl.ds(DY - L, L)]
                        vals = jnp.where(iota < 8, cnt_pat, v984)
                        plsc.addupdate_scatter(acc, [row, tcols], vals)

        npairs = jnp.maximum((nchunks + 1) // 2, 1)
        start(0, lb0, rb0, sem0)
        start(1, lb1, rb1, sem1)

        def pair(p, _):
            c0 = 2 * p
            drain(lb0, rb0, sem0)
            process(c0, lb0, rb0)
            start(c0 + 2, lb0, rb0, sem0)
            drain(lb1, rb1, sem1)
            process(c0 + 1, lb1, rb1)
            start(c0 + 3, lb1, rb1, sem1)
            return 0

        lax.fori_loop(0, npairs, pair, 0)
        drain(lb0, rb0, sem0)
        drain(lb1, rb1, sem1)

        out0 = wid * SEG_PER_W
        pltpu.sync_copy(acc.at[pl.ds(0, SEG_PER_W)],
                        out_hbm.at[pl.ds(out0, SEG_PER_W)])

    return body


def _segsum_call(rows, labels, starts, dsrc, dacc, ch, with_count):
    run = pl.kernel(
        _make_segsum_body(dsrc, dacc, ch, with_count),
        out_type=jax.ShapeDtypeStruct((NSEG, dacc), jnp.float32),
        mesh=plsc.VectorSubcoreMesh(
            core_axis_name="c", subcore_axis_name="s", num_cores=NC,
            num_subcores=NS),
        compiler_params=pltpu.CompilerParams(needs_layout_passes=False),
        scratch_types=[
            pltpu.VMEM((L,), jnp.int32),
            pltpu.VMEM((ch,), jnp.int32),
            pltpu.VMEM((ch,), jnp.int32),
            pltpu.VMEM((ch, dsrc), jnp.float32),
            pltpu.VMEM((ch, dsrc), jnp.float32),
            pltpu.VMEM((SEG_PER_W + 1, dacc), jnp.float32),
            pltpu.SemaphoreType.DMA,
            pltpu.SemaphoreType.DMA,
        ],
    )
    return run(rows, labels, starts)


def _final_body(suma_ref, sumy_ref, w2_ref, b2_ref, out_ref):
    sa = suma_ref[...]
    sy = sumy_ref[:, 0:DY]
    cnt = sumy_ref[:, DY:DY + 1]
    valid = cnt > 0.0
    safe = jnp.where(valid, cnt, 1.0)
    res = sa / safe
    logits = lax.dot_general(
        res, w2_ref[...], (((1,), (1,)), ((), ())),
        preferred_element_type=jnp.float32) + b2_ref[...]
    m = jnp.max(logits, axis=1, keepdims=True)
    e = jnp.exp(logits - m)
    p = e / jnp.sum(e, axis=1, keepdims=True)
    m2 = jnp.max(p, axis=1, keepdims=True)
    lse = jnp.log(jnp.sum(jnp.exp(p - m2), axis=1, keepdims=True)) + m2
    logp = p - lse
    per = jnp.sum(sy * logp, axis=1, keepdims=True) / safe
    per = jnp.where(valid, per, 0.0)
    u = jnp.sum(valid.astype(jnp.float32), axis=0, keepdims=True)
    out_ref[...] = -jnp.sum(per, axis=0, keepdims=True) / u


def _final_call(suma, sumy, W2, b2):
    full = lambda: (0, 0)
    return pl.pallas_call(
        _final_body,
        in_specs=[
            pl.BlockSpec((NSEG, D1), full),
            pl.BlockSpec((NSEG, DYA), full),
            pl.BlockSpec((DY, D1), full),
            pl.BlockSpec((1, DY), full),
        ],
        out_specs=pl.BlockSpec((1, 1), full),
        out_shape=jax.ShapeDtypeStruct((1, 1), jnp.float32),
    )(suma, sumy, W2, b2)


def kernel(x1, x2, y, W1, b1, W2, b2, g1, be1, g2, be2, labels):
    labels = labels.astype(jnp.int32)
    bounds = jnp.searchsorted(
        labels, jnp.arange(0, NSEG + 1, SEG_PER_W, dtype=jnp.int32)
    ).astype(jnp.int32)
    starts = jnp.zeros((NW, L), jnp.int32)
    starts = starts.at[:, 0].set(bounds[:-1]).at[:, 1].set(bounds[1:])
    starts = starts.reshape(NW * L)
    sumy = _segsum_call(y, labels, starts, DY, DYA, CH, True)
    a = _gate_call(
        x1, x2, W1, b1.reshape(1, 1), g1.reshape(1, D1), be1.reshape(1, D1),
        g2.reshape(1, D1), be2.reshape(1, D1))
    suma = _segsum_call(a, labels, starts, D1, D1, 2 * CH, False)
    out = _final_call(suma, sumy, W2, b2.reshape(1, DY))
    return out[0, 0]
